# Initial kernel scaffold; baseline (speedup 1.0000x reference)
import jax
import jax.numpy as jnp
from jax import lax
from jax.experimental import pallas as pl
from jax.experimental.pallas import tpu as pltpu
from jax.experimental.pallas import tpu_sc as plsc

N, E, D, H, C, G = 10000, 160000, 256, 256, 10, 64
NP = 10240
NT = 16
CHUNK = 128
NCH = 79
EP = NT * NCH * CHUNK
EW_PER_W = EP // 32
ROWS_PER_TILE = NP // NT
R = 1024
GRID = NP // R


def _mesh():
    return plsc.VectorSubcoreMesh(core_axis_name="c", subcore_axis_name="s")


def _mp_body(ta, tb, src3, dst3, outa, outb, acc_sh, src_v, dst_v, rows_v,
             zero_v, sem):
    c = lax.axis_index("c")
    s = lax.axis_index("s")

    pltpu.sync_copy(src3.at[s], src_v)
    pltpu.sync_copy(dst3.at[s], dst_v)

    for i in range(16):
        for j in range(8):
            zero_v[i, pl.ds(j * 16, 16)] = jnp.zeros((16,), jnp.float32)

    def zloop(r, _):
        pltpu.sync_copy(zero_v, acc_sh.at[pl.ds(s * ROWS_PER_TILE + r * 16, 16)])
        return 0

    lax.fori_loop(0, ROWS_PER_TILE // 16, zloop, 0)
    plsc.subcore_barrier()

    def run(tab, out):
        def body(j, _):
            pltpu.async_copy(tab.at[src_v.at[j]], rows_v, sem).wait()
            pltpu.sync_copy(rows_v, acc_sh.at[dst_v.at[j]], add=True)
            return 0

        lax.fori_loop(0, NCH, body, 0)
        plsc.subcore_barrier()
        pltpu.sync_copy(acc_sh.at[pl.ds(s * ROWS_PER_TILE, ROWS_PER_TILE)],
                        out.at[pl.ds(s * ROWS_PER_TILE, ROWS_PER_TILE)])

    @pl.when(c == 0)
    def _():
        run(ta, outa)

    @pl.when(c == 1)
    def _():
        run(tb, outb)


@jax.jit
def _mp(ta, tb, src3, dst3):
    return pl.kernel(
        _mp_body,
        out_type=(
            jax.ShapeDtypeStruct((NP, 128), jnp.float32),
            jax.ShapeDtypeStruct((NP, 128), jnp.float32),
        ),
        mesh=_mesh(),
        scratch_types=[
            pltpu.VMEM_SHARED((NP, 128), jnp.float32),
            pltpu.VMEM((NCH, CHUNK), jnp.int32),
            pltpu.VMEM((NCH, CHUNK), jnp.int32),
            pltpu.VMEM((CHUNK, 128), jnp.float32),
            pltpu.VMEM((16, 128), jnp.float32),
            pltpu.SemaphoreType.DMA,
        ],
    )(ta, tb, src3, dst3)


def _ew_body(att, src2, dst2, ew, att_v, s_v, d_v, o_v):
    c = lax.axis_index("c")
    s = lax.axis_index("s")
    w = s * 2 + c

    pltpu.sync_copy(att, att_v)
    pltpu.sync_copy(src2.at[w], s_v)
    pltpu.sync_copy(dst2.at[w], d_v)

    def body(i, _):
        si = s_v[pl.ds(i * 16, 16)]
        di = d_v[pl.ds(i * 16, 16)]
        a = plsc.load_gather(att_v, [si])
        b = plsc.load_gather(att_v, [di])
        o_v[pl.ds(i * 16, 16)] = a * b
        return 0

    lax.fori_loop(0, EW_PER_W // 16, body, 0)
    pltpu.sync_copy(o_v, ew.at[pl.ds(w * EW_PER_W, EW_PER_W)])


@jax.jit
def _ew(att, src2, dst2):
    return pl.kernel(
        _ew_body,
        out_type=jax.ShapeDtypeStruct((EP,), jnp.float32),
        mesh=_mesh(),
        scratch_types=[
            pltpu.VMEM((NP,), jnp.float32),
            pltpu.VMEM((EW_PER_W,), jnp.int32),
            pltpu.VMEM((EW_PER_W,), jnp.int32),
            pltpu.VMEM((EW_PER_W,), jnp.float32),
        ],
    )(att, src2, dst2)


def _tca_body(acc1a, acc1b, xr, bf, gum, Wg, bg, Wib, bib,
              asn_o, u0a_o, u0b_o, u1a_o, u1b_o, g1_o):
    i = pl.program_id(0)
    agg = jnp.concatenate([acc1a[...], acc1b[...]], axis=1)
    z1 = jnp.maximum(agg @ Wg[...] + bg[...], 0.0)
    score = z1 @ Wib[...] + bib[...]
    m = jnp.max(score, axis=1, keepdims=True)
    e = jnp.exp(score - m)
    a1 = e / jnp.sum(e, axis=1, keepdims=True)
    t = a1 + gum[...]
    m2 = jnp.max(t, axis=1, keepdims=True)
    e2 = jnp.exp(t - m2)
    asn = e2 / jnp.sum(e2, axis=1, keepdims=True)
    asn_o[...] = asn
    x = xr[...]
    u0 = asn[:, 0:1] * x
    u1 = asn[:, 1:2] * x
    u0a_o[...] = u0[:, :128]
    u0b_o[...] = u0[:, 128:]
    u1a_o[...] = u1[:, :128]
    u1b_o[...] = u1[:, 128:]
    oh = jnp.where(bf[...] == lax.broadcasted_iota(jnp.float32, (R, G), 1),
                   1.0, 0.0)
    contrib = lax.dot_general(oh, z1, (((0,), (0,)), ((), ())),
                              preferred_element_type=jnp.float32)

    @pl.when(i == 0)
    def _():
        g1_o[...] = jnp.zeros_like(g1_o)

    g1_o[...] += contrib


@jax.jit
def _tca(acc1a, acc1b, xp, bf, gum, Wg, bg, Wib, bib):
    row = lambda i: (i, 0)
    const = lambda i: (0, 0)
    return pl.pallas_call(
        _tca_body,
        grid=(GRID,),
        in_specs=[
            pl.BlockSpec((R, 128), row),
            pl.BlockSpec((R, 128), row),
            pl.BlockSpec((R, D), row),
            pl.BlockSpec((R, 1), row),
            pl.BlockSpec((R, 2), row),
            pl.BlockSpec((D, H), const),
            pl.BlockSpec((1, H), const),
            pl.BlockSpec((H, 2), const),
            pl.BlockSpec((1, 2), const),
        ],
        out_specs=[
            pl.BlockSpec((R, 2), row),
            pl.BlockSpec((R, 128), row),
            pl.BlockSpec((R, 128), row),
            pl.BlockSpec((R, 128), row),
            pl.BlockSpec((R, 128), row),
            pl.BlockSpec((G, H), const),
        ],
        out_shape=[
            jax.ShapeDtypeStruct((NP, 2), jnp.float32),
            jax.ShapeDtypeStruct((NP, 128), jnp.float32),
            jax.ShapeDtypeStruct((NP, 128), jnp.float32),
            jax.ShapeDtypeStruct((NP, 128), jnp.float32),
            jax.ShapeDtypeStruct((NP, 128), jnp.float32),
            jax.ShapeDtypeStruct((G, H), jnp.float32),
        ],
    )(acc1a, acc1b, xp, bf, gum, Wg, bg, Wib, bib)


def _tcb_body(accAa, accAb, accBa, accBb, asn, bf, Wctx, bctx, Wobj, bobj,
              g1r, Pr, yr, Wm1, bm1, Wm3, bm3, Wco1, bco1, Wco2, bco2,
              Wo1, bo1, Wo2, bo2, Wc1, bc1, Wc2, bc2, Wcs1, bcs1, Wcs2, bcs2,
              gM_o, gR_o, h1_o, hM_o, p1_o, pM_o, hco_o, hres_o, hcos_o, ys_o):
    i = pl.program_id(0)
    a = asn[...]
    aggO = a[:, 0:1] * jnp.concatenate([accAa[...], accAb[...]], axis=1)
    aggC = a[:, 1:2] * jnp.concatenate([accBa[...], accBb[...]], axis=1)
    zM = jnp.maximum(aggO @ Wctx[...] + bctx[...], 0.0)
    zR = jnp.maximum(aggC @ Wobj[...] + bobj[...], 0.0)
    oh = jnp.where(bf[...] == lax.broadcasted_iota(jnp.float32, (R, G), 1),
                   1.0, 0.0)
    cM = lax.dot_general(oh, zM, (((0,), (0,)), ((), ())),
                         preferred_element_type=jnp.float32)
    cR = lax.dot_general(oh, zR, (((0,), (0,)), ((), ())),
                         preferred_element_type=jnp.float32)

    @pl.when(i == 0)
    def _():
        gM_o[...] = jnp.zeros_like(gM_o)
        gR_o[...] = jnp.zeros_like(gR_o)

    gM_o[...] += cM
    gR_o[...] += cR

    @pl.when(i == GRID - 1)
    def _():
        gM = gM_o[...]
        gR = gR_o[...]
        g1 = g1r[...]
        P = Pr[...]
        g_co = P @ gR + gM
        g_co_s = gR + P @ gM
        relu = lambda v: jnp.maximum(v, 0.0)
        h1_o[...] = g1 @ Wm1[...] + bm1[...]
        hM_o[...] = relu(gM @ Wo1[...] + bo1[...]) @ Wo2[...] + bo2[...]
        p1_o[...] = g1 @ Wm3[...] + bm3[...]
        pM_o[...] = gM @ Wm3[...] + bm3[...]
        hco_o[...] = relu(g_co @ Wco1[...] + bco1[...]) @ Wco2[...] + bco2[...]
        hres_o[...] = relu(gR @ Wc1[...] + bc1[...]) @ Wc2[...] + bc2[...]
        hcos_o[...] = relu(g_co_s @ Wcs1[...] + bcs1[...]) @ Wcs2[...] + bcs2[...]
        ys_o[...] = P @ yr[...]


@jax.jit
def _tcb(accAa, accAb, accBa, accBb, asn, bf, Wctx, bctx, Wobj, bobj,
         g1, P, yf, Wm1, bm1, Wm3, bm3, Wco1, bco1, Wco2, bco2,
         Wo1, bo1, Wo2, bo2, Wc1, bc1, Wc2, bc2, Wcs1, bcs1, Wcs2, bcs2):
    row = lambda i: (i, 0)
    const = lambda i: (0, 0)
    wspec = lambda shape: pl.BlockSpec(shape, const)
    return pl.pallas_call(
        _tcb_body,
        grid=(GRID,),
        in_specs=[
            pl.BlockSpec((R, 128), row),
            pl.BlockSpec((R, 128), row),
            pl.BlockSpec((R, 128), row),
            pl.BlockSpec((R, 128), row),
            pl.BlockSpec((R, 2), row),
            pl.BlockSpec((R, 1), row),
            wspec((D, H)), wspec((1, H)), wspec((D, H)), wspec((1, H)),
            wspec((G, H)), wspec((G, G)), wspec((G, 1)),
            wspec((H, C)), wspec((1, C)),
            wspec((H, H)), wspec((1, H)),
            wspec((H, H)), wspec((1, H)), wspec((H, C)), wspec((1, C)),
            wspec((H, H)), wspec((1, H)), wspec((H, C)), wspec((1, C)),
            wspec((H, H)), wspec((1, H)), wspec((H, C)), wspec((1, C)),
            wspec((H, H)), wspec((1, H)), wspec((H, C)), wspec((1, C)),
        ],
        out_specs=[
            wspec((G, H)), wspec((G, H)),
            wspec((G, C)), wspec((G, C)), wspec((G, H)), wspec((G, H)),
            wspec((G, C)), wspec((G, C)), wspec((G, C)), wspec((G, 1)),
        ],
        out_shape=[
            jax.ShapeDtypeStruct((G, H), jnp.float32),
            jax.ShapeDtypeStruct((G, H), jnp.float32),
            jax.ShapeDtypeStruct((G, C), jnp.float32),
            jax.ShapeDtypeStruct((G, C), jnp.float32),
            jax.ShapeDtypeStruct((G, H), jnp.float32),
            jax.ShapeDtypeStruct((G, H), jnp.float32),
            jax.ShapeDtypeStruct((G, C), jnp.float32),
            jax.ShapeDtypeStruct((G, C), jnp.float32),
            jax.ShapeDtypeStruct((G, C), jnp.float32),
            jax.ShapeDtypeStruct((G, 1), jnp.float32),
        ],
    )(accAa, accAb, accBa, accBb, asn, bf, Wctx, bctx, Wobj, bobj,
      g1, P, yf, Wm1, bm1, Wm3, bm3, Wco1, bco1, Wco2, bco2,
      Wo1, bo1, Wo2, bo2, Wc1, bc1, Wc2, bc2, Wcs1, bcs1, Wcs2, bcs2)


def kernel(x, edge_index, batch, y, W_g1, b_g1, W_ctx, b_ctx, W_obj, b_obj,
           W_ib, b_ib, W_m1, b_m1, W_m3, b_m3, W_co1, b_co1, W_co2, b_co2,
           W_o1, b_o1, W_o2, b_o2, W_c1, b_c1, W_c2, b_c2, W_cs1, b_cs1,
           W_cs2, b_cs2):
    f32 = jnp.float32

    xp = jnp.pad(x, ((0, NP - N), (0, 0)))
    xa = xp[:, :128]
    xb = xp[:, 128:]

    src = edge_index[0]
    dst = edge_index[1]
    src_p = jnp.pad(src, (0, EP - E))
    dst_p = jnp.pad(dst, (0, EP - E), constant_values=N)
    src3 = src_p.reshape(NT, NCH, CHUNK)
    dst3 = dst_p.reshape(NT, NCH, CHUNK)
    src2 = src_p.reshape(32, EW_PER_W)
    dst2 = dst_p.reshape(32, EW_PER_W)

    bf = jnp.pad(batch, (0, NP - N), constant_values=-1)
    bf = bf.astype(f32).reshape(NP, 1)

    u = jax.random.uniform(jax.random.key(42), (N, 2), minval=1e-10,
                           maxval=1.0)
    gum = -jnp.log(-jnp.log(u))
    gum = jnp.pad(gum, ((0, NP - N), (0, 0)))

    random_idx = jax.random.permutation(jax.random.key(7), G)
    P = (random_idx[:, None] == jnp.arange(G)[None, :]).astype(f32)
    yf = y.astype(f32)

    rb = lambda b: b.reshape(1, -1)

    acc1a, acc1b = _mp(xa, xb, src3, dst3)

    asn, u0a, u0b, u1a, u1b, g1 = _tca(
        acc1a, acc1b, xp, bf, gum, W_g1, rb(b_g1), W_ib, rb(b_ib))

    accAa, accAb = _mp(u0a, u0b, src3, dst3)
    accBa, accBb = _mp(u1a, u1b, src3, dst3)

    att0 = jnp.ascontiguousarray(asn[:, 0])
    ew = _ew(att0, src2, dst2)

    (gM, gR, h1, hM, p1, pM, hco, hres, hcos, ysf) = _tcb(
        accAa, accAb, accBa, accBb, asn, bf, W_ctx, rb(b_ctx), W_obj,
        rb(b_obj), g1, P, yf, W_m1, rb(b_m1), W_m3, rb(b_m3),
        W_co1, rb(b_co1), W_co2, rb(b_co2), W_o1, rb(b_o1), W_o2, rb(b_o2),
        W_c1, rb(b_c1), W_c2, rb(b_c2), W_cs1, rb(b_cs1), W_cs2, rb(b_cs2))

    assignment = asn[:N]
    edge_weight_o = ew[:E].reshape(E, 1)
    y_shuf = ysf[:, 0].astype(jnp.int32)

    return (h1, g1, hM, gM, p1, pM, assignment, hco, hres, edge_weight_o,
            hcos, y_shuf)


# speedup vs baseline: 7.1229x; 7.1229x over previous
import jax
import jax.numpy as jnp
from jax import lax
from jax.experimental import pallas as pl
from jax.experimental.pallas import tpu as pltpu
from jax.experimental.pallas import tpu_sc as plsc

N, E, D, H, C, G = 10000, 160000, 256, 256, 10, 64
NP = 10240
NT = 16
CHUNK = 128
NCH = 79
EP = NT * NCH * CHUNK
EW_NCH = 40
EW_PER_W = EW_NCH * CHUNK
EP2 = 32 * EW_PER_W
ROWS_PER_TILE = NP // NT
R = 1024
GRID = NP // R


def _mesh():
    return plsc.VectorSubcoreMesh(core_axis_name="c", subcore_axis_name="s")


def _mp_body(ta, tb, src3, dst3, outa, outb, acc_sh, src_v, dst_v, rows_v,
             zero_v, sem):
    c = lax.axis_index("c")
    s = lax.axis_index("s")

    pltpu.sync_copy(src3.at[s], src_v)
    pltpu.sync_copy(dst3.at[s], dst_v)

    for i in range(16):
        for j in range(8):
            zero_v[i, pl.ds(j * 16, 16)] = jnp.zeros((16,), jnp.float32)

    def zloop(r, _):
        pltpu.sync_copy(zero_v, acc_sh.at[pl.ds(s * ROWS_PER_TILE + r * 16, 16)])
        return 0

    lax.fori_loop(0, ROWS_PER_TILE // 16, zloop, 0)
    plsc.subcore_barrier()

    def run(tab, out):
        def body(j, _):
            pltpu.async_copy(tab.at[src_v.at[j]], rows_v, sem).wait()
            pltpu.sync_copy(rows_v, acc_sh.at[dst_v.at[j]], add=True)
            return 0

        lax.fori_loop(0, NCH, body, 0)
        plsc.subcore_barrier()
        pltpu.sync_copy(acc_sh.at[pl.ds(s * ROWS_PER_TILE, ROWS_PER_TILE)],
                        out.at[pl.ds(s * ROWS_PER_TILE, ROWS_PER_TILE)])

    @pl.when(c == 0)
    def _():
        run(ta, outa)

    @pl.when(c == 1)
    def _():
        run(tb, outb)


@jax.jit
def _mp(ta, tb, src3, dst3):
    return pl.kernel(
        _mp_body,
        out_type=(
            jax.ShapeDtypeStruct((NP, 128), jnp.float32),
            jax.ShapeDtypeStruct((NP, 128), jnp.float32),
        ),
        mesh=_mesh(),
        scratch_types=[
            pltpu.VMEM_SHARED((NP, 128), jnp.float32),
            pltpu.VMEM((NCH, CHUNK), jnp.int32),
            pltpu.VMEM((NCH, CHUNK), jnp.int32),
            pltpu.VMEM((CHUNK, 128), jnp.float32),
            pltpu.VMEM((16, 128), jnp.float32),
            pltpu.SemaphoreType.DMA,
        ],
    )(ta, tb, src3, dst3)


def _ew_body(att, src2, dst2, ew, s_v, d_v, a_v, b_v, o_v, sem_a, sem_b):
    c = lax.axis_index("c")
    s = lax.axis_index("s")
    w = c * 16 + s

    pltpu.sync_copy(src2.at[w], s_v)
    pltpu.sync_copy(dst2.at[w], d_v)

    def body(j, _):
        cp_a = pltpu.async_copy(att.at[s_v.at[j]], a_v, sem_a)
        cp_b = pltpu.async_copy(att.at[d_v.at[j]], b_v, sem_b)
        cp_a.wait()
        cp_b.wait()
        for k in range(CHUNK // 16):
            o_v[j, pl.ds(k * 16, 16)] = (a_v[pl.ds(k * 16, 16)] *
                                         b_v[pl.ds(k * 16, 16)])
        return 0

    lax.fori_loop(0, EW_NCH, body, 0)
    pltpu.sync_copy(o_v, ew.at[pl.ds(w * EW_NCH, EW_NCH)])


@jax.jit
def _ew(att, src2, dst2):
    return pl.kernel(
        _ew_body,
        out_type=jax.ShapeDtypeStruct((32 * EW_NCH, CHUNK), jnp.float32),
        mesh=_mesh(),
        scratch_types=[
            pltpu.VMEM((EW_NCH, CHUNK), jnp.int32),
            pltpu.VMEM((EW_NCH, CHUNK), jnp.int32),
            pltpu.VMEM((CHUNK,), jnp.float32),
            pltpu.VMEM((CHUNK,), jnp.float32),
            pltpu.VMEM((EW_NCH, CHUNK), jnp.float32),
            pltpu.SemaphoreType.DMA,
            pltpu.SemaphoreType.DMA,
        ],
    )(att, src2, dst2)


def _tca_body(acc1a, acc1b, xr, bf, gum, Wg, bg, Wib, bib,
              asn_o, u0a_o, u0b_o, u1a_o, u1b_o, g1_o):
    i = pl.program_id(0)
    agg = jnp.concatenate([acc1a[...], acc1b[...]], axis=1)
    z1 = jnp.maximum(agg @ Wg[...] + bg[...], 0.0)
    score = z1 @ Wib[...] + bib[...]
    m = jnp.max(score, axis=1, keepdims=True)
    e = jnp.exp(score - m)
    a1 = e / jnp.sum(e, axis=1, keepdims=True)
    t = a1 + gum[...]
    m2 = jnp.max(t, axis=1, keepdims=True)
    e2 = jnp.exp(t - m2)
    asn = e2 / jnp.sum(e2, axis=1, keepdims=True)
    asn_o[...] = asn
    x = xr[...]
    u0 = asn[:, 0:1] * x
    u1 = asn[:, 1:2] * x
    u0a_o[...] = u0[:, :128]
    u0b_o[...] = u0[:, 128:]
    u1a_o[...] = u1[:, :128]
    u1b_o[...] = u1[:, 128:]
    oh = jnp.where(bf[...] == lax.broadcasted_iota(jnp.int32, (R, G), 1),
                   1.0, 0.0)
    contrib = lax.dot_general(oh, z1, (((0,), (0,)), ((), ())),
                              preferred_element_type=jnp.float32)

    @pl.when(i == 0)
    def _():
        g1_o[...] = jnp.zeros_like(g1_o)

    g1_o[...] += contrib


@jax.jit
def _tca(acc1a, acc1b, xp, bf, gum, Wg, bg, Wib, bib):
    row = lambda i: (i, 0)
    const = lambda i: (0, 0)
    return pl.pallas_call(
        _tca_body,
        grid=(GRID,),
        in_specs=[
            pl.BlockSpec((R, 128), row),
            pl.BlockSpec((R, 128), row),
            pl.BlockSpec((R, D), row),
            pl.BlockSpec((R, 1), row),
            pl.BlockSpec((R, 2), row),
            pl.BlockSpec((D, H), const),
            pl.BlockSpec((1, H), const),
            pl.BlockSpec((H, 2), const),
            pl.BlockSpec((1, 2), const),
        ],
        out_specs=[
            pl.BlockSpec((R, 2), row),
            pl.BlockSpec((R, 128), row),
            pl.BlockSpec((R, 128), row),
            pl.BlockSpec((R, 128), row),
            pl.BlockSpec((R, 128), row),
            pl.BlockSpec((G, H), const),
        ],
        out_shape=[
            jax.ShapeDtypeStruct((NP, 2), jnp.float32),
            jax.ShapeDtypeStruct((NP, 128), jnp.float32),
            jax.ShapeDtypeStruct((NP, 128), jnp.float32),
            jax.ShapeDtypeStruct((NP, 128), jnp.float32),
            jax.ShapeDtypeStruct((NP, 128), jnp.float32),
            jax.ShapeDtypeStruct((G, H), jnp.float32),
        ],
    )(acc1a, acc1b, xp, bf, gum, Wg, bg, Wib, bib)


def _tcb_body(accAa, accAb, accBa, accBb, asn, bf, Wctx, bctx, Wobj, bobj,
              g1r, Pr, yr, Wm1, bm1, Wm3, bm3, Wco1, bco1, Wco2, bco2,
              Wo1, bo1, Wo2, bo2, Wc1, bc1, Wc2, bc2, Wcs1, bcs1, Wcs2, bcs2,
              gM_o, gR_o, h1_o, hM_o, p1_o, pM_o, hco_o, hres_o, hcos_o, ys_o):
    i = pl.program_id(0)
    a = asn[...]
    aggO = a[:, 0:1] * jnp.concatenate([accAa[...], accAb[...]], axis=1)
    aggC = a[:, 1:2] * jnp.concatenate([accBa[...], accBb[...]], axis=1)
    zM = jnp.maximum(aggO @ Wctx[...] + bctx[...], 0.0)
    zR = jnp.maximum(aggC @ Wobj[...] + bobj[...], 0.0)
    oh = jnp.where(bf[...] == lax.broadcasted_iota(jnp.int32, (R, G), 1),
                   1.0, 0.0)
    cM = lax.dot_general(oh, zM, (((0,), (0,)), ((), ())),
                         preferred_element_type=jnp.float32)
    cR = lax.dot_general(oh, zR, (((0,), (0,)), ((), ())),
                         preferred_element_type=jnp.float32)

    @pl.when(i == 0)
    def _():
        gM_o[...] = jnp.zeros_like(gM_o)
        gR_o[...] = jnp.zeros_like(gR_o)

    gM_o[...] += cM
    gR_o[...] += cR

    @pl.when(i == GRID - 1)
    def _():
        gM = gM_o[...]
        gR = gR_o[...]
        g1 = g1r[...]
        P = Pr[...]
        g_co = P @ gR + gM
        g_co_s = gR + P @ gM
        relu = lambda v: jnp.maximum(v, 0.0)
        h1_o[...] = g1 @ Wm1[...] + bm1[...]
        hM_o[...] = relu(gM @ Wo1[...] + bo1[...]) @ Wo2[...] + bo2[...]
        p1_o[...] = g1 @ Wm3[...] + bm3[...]
        pM_o[...] = gM @ Wm3[...] + bm3[...]
        hco_o[...] = relu(g_co @ Wco1[...] + bco1[...]) @ Wco2[...] + bco2[...]
        hres_o[...] = relu(gR @ Wc1[...] + bc1[...]) @ Wc2[...] + bc2[...]
        hcos_o[...] = relu(g_co_s @ Wcs1[...] + bcs1[...]) @ Wcs2[...] + bcs2[...]
        ys_o[...] = P @ yr[...]


@jax.jit
def _tcb(accAa, accAb, accBa, accBb, asn, bf, Wctx, bctx, Wobj, bobj,
         g1, P, yf, Wm1, bm1, Wm3, bm3, Wco1, bco1, Wco2, bco2,
         Wo1, bo1, Wo2, bo2, Wc1, bc1, Wc2, bc2, Wcs1, bcs1, Wcs2, bcs2):
    row = lambda i: (i, 0)
    const = lambda i: (0, 0)
    wspec = lambda shape: pl.BlockSpec(shape, const)
    return pl.pallas_call(
        _tcb_body,
        grid=(GRID,),
        in_specs=[
            pl.BlockSpec((R, 128), row),
            pl.BlockSpec((R, 128), row),
            pl.BlockSpec((R, 128), row),
            pl.BlockSpec((R, 128), row),
            pl.BlockSpec((R, 2), row),
            pl.BlockSpec((R, 1), row),
            wspec((D, H)), wspec((1, H)), wspec((D, H)), wspec((1, H)),
            wspec((G, H)), wspec((G, G)), wspec((G, 1)),
            wspec((H, C)), wspec((1, C)),
            wspec((H, H)), wspec((1, H)),
            wspec((H, H)), wspec((1, H)), wspec((H, C)), wspec((1, C)),
            wspec((H, H)), wspec((1, H)), wspec((H, C)), wspec((1, C)),
            wspec((H, H)), wspec((1, H)), wspec((H, C)), wspec((1, C)),
            wspec((H, H)), wspec((1, H)), wspec((H, C)), wspec((1, C)),
        ],
        out_specs=[
            wspec((G, H)), wspec((G, H)),
            wspec((G, C)), wspec((G, C)), wspec((G, H)), wspec((G, H)),
            wspec((G, C)), wspec((G, C)), wspec((G, C)), wspec((G, 1)),
        ],
        out_shape=[
            jax.ShapeDtypeStruct((G, H), jnp.float32),
            jax.ShapeDtypeStruct((G, H), jnp.float32),
            jax.ShapeDtypeStruct((G, C), jnp.float32),
            jax.ShapeDtypeStruct((G, C), jnp.float32),
            jax.ShapeDtypeStruct((G, H), jnp.float32),
            jax.ShapeDtypeStruct((G, H), jnp.float32),
            jax.ShapeDtypeStruct((G, C), jnp.float32),
            jax.ShapeDtypeStruct((G, C), jnp.float32),
            jax.ShapeDtypeStruct((G, C), jnp.float32),
            jax.ShapeDtypeStruct((G, 1), jnp.float32),
        ],
    )(accAa, accAb, accBa, accBb, asn, bf, Wctx, bctx, Wobj, bobj,
      g1, P, yf, Wm1, bm1, Wm3, bm3, Wco1, bco1, Wco2, bco2,
      Wo1, bo1, Wo2, bo2, Wc1, bc1, Wc2, bc2, Wcs1, bcs1, Wcs2, bcs2)


def kernel(x, edge_index, batch, y, W_g1, b_g1, W_ctx, b_ctx, W_obj, b_obj,
           W_ib, b_ib, W_m1, b_m1, W_m3, b_m3, W_co1, b_co1, W_co2, b_co2,
           W_o1, b_o1, W_o2, b_o2, W_c1, b_c1, W_c2, b_c2, W_cs1, b_cs1,
           W_cs2, b_cs2):
    f32 = jnp.float32

    xp = jnp.pad(x, ((0, NP - N), (0, 0)))
    xa = xp[:, :128]
    xb = xp[:, 128:]

    src = edge_index[0]
    dst = edge_index[1]
    src_p = jnp.pad(src, (0, EP - E))
    dst_p = jnp.pad(dst, (0, EP - E), constant_values=N)
    src3 = src_p.reshape(NT, NCH, CHUNK)
    dst3 = dst_p.reshape(NT, NCH, CHUNK)
    src2 = jnp.pad(src, (0, EP2 - E)).reshape(32, EW_NCH, CHUNK)
    dst2 = jnp.pad(dst, (0, EP2 - E)).reshape(32, EW_NCH, CHUNK)

    bf = jnp.pad(batch, (0, NP - N), constant_values=-1).reshape(NP, 1)

    u = jax.random.uniform(jax.random.key(42), (N, 2), minval=1e-10,
                           maxval=1.0)
    gum = -jnp.log(-jnp.log(u))
    gum = jnp.pad(gum, ((0, NP - N), (0, 0)))

    random_idx = jax.random.permutation(jax.random.key(7), G)
    P = (random_idx[:, None] == jnp.arange(G)[None, :]).astype(f32)
    yf = y.astype(f32)

    rb = lambda b: b.reshape(1, -1)

    acc1a, acc1b = _mp(xa, xb, src3, dst3)

    asn, u0a, u0b, u1a, u1b, g1 = _tca(
        acc1a, acc1b, xp, bf, gum, W_g1, rb(b_g1), W_ib, rb(b_ib))

    accAa, accAb = _mp(u0a, u0b, src3, dst3)
    accBa, accBb = _mp(u1a, u1b, src3, dst3)

    att0 = asn[:, 0]
    ew = _ew(att0, src2, dst2)

    (gM, gR, h1, hM, p1, pM, hco, hres, hcos, ysf) = _tcb(
        accAa, accAb, accBa, accBb, asn, bf, W_ctx, rb(b_ctx), W_obj,
        rb(b_obj), g1, P, yf, W_m1, rb(b_m1), W_m3, rb(b_m3),
        W_co1, rb(b_co1), W_co2, rb(b_co2), W_o1, rb(b_o1), W_o2, rb(b_o2),
        W_c1, rb(b_c1), W_c2, rb(b_c2), W_cs1, rb(b_cs1), W_cs2, rb(b_cs2))

    assignment = asn[:N]
    edge_weight_o = ew.reshape(-1)[:E].reshape(E, 1)
    y_shuf = ysf[:, 0].astype(jnp.int32)

    return (h1, g1, hM, gM, p1, pM, assignment, hco, hres, edge_weight_o,
            hcos, y_shuf)


# speedup vs baseline: 9.7824x; 1.3734x over previous
import jax
import jax.numpy as jnp
from jax import lax
from jax.experimental import pallas as pl
from jax.experimental.pallas import tpu as pltpu
from jax.experimental.pallas import tpu_sc as plsc

N, E, D, H, C, G = 10000, 160000, 256, 256, 10, 64
NP = 10240
NT = 16
CHUNK = 128
NCH = 79
EP = NT * NCH * CHUNK
EW_NCH = 40
EW_PER_W = EW_NCH * CHUNK
EP2 = 32 * EW_PER_W
ROWS_PER_TILE = NP // NT
R = 1024
GRID = NP // R


def _mesh():
    return plsc.VectorSubcoreMesh(core_axis_name="c", subcore_axis_name="s")


def _mp_body(ta, tb, src3, dst3, outa, outb, acc_sh, src_v, dst_v, rows_v,
             zero_v, sem):
    c = lax.axis_index("c")
    s = lax.axis_index("s")

    pltpu.sync_copy(src3.at[s], src_v)
    pltpu.sync_copy(dst3.at[s], dst_v)

    for i in range(16):
        for j in range(8):
            zero_v[i, pl.ds(j * 16, 16)] = jnp.zeros((16,), jnp.float32)

    def zloop(r, _):
        pltpu.sync_copy(zero_v, acc_sh.at[pl.ds(s * ROWS_PER_TILE + r * 16, 16)])
        return 0

    lax.fori_loop(0, ROWS_PER_TILE // 16, zloop, 0)
    plsc.subcore_barrier()

    def run(tab, out):
        def body(j, _):
            pltpu.async_copy(tab.at[src_v.at[j]], rows_v, sem).wait()
            pltpu.sync_copy(rows_v, acc_sh.at[dst_v.at[j]], add=True)
            return 0

        lax.fori_loop(0, NCH, body, 0)
        plsc.subcore_barrier()
        pltpu.sync_copy(acc_sh.at[pl.ds(s * ROWS_PER_TILE, ROWS_PER_TILE)],
                        out.at[pl.ds(s * ROWS_PER_TILE, ROWS_PER_TILE)])

    @pl.when(c == 0)
    def _():
        run(ta, outa)

    @pl.when(c == 1)
    def _():
        run(tb, outb)


@jax.jit
def _mp(ta, tb, src3, dst3):
    return pl.kernel(
        _mp_body,
        out_type=(
            jax.ShapeDtypeStruct((NP, 128), jnp.float32),
            jax.ShapeDtypeStruct((NP, 128), jnp.float32),
        ),
        mesh=_mesh(),
        scratch_types=[
            pltpu.VMEM_SHARED((NP, 128), jnp.float32),
            pltpu.VMEM((NCH, CHUNK), jnp.int32),
            pltpu.VMEM((NCH, CHUNK), jnp.int32),
            pltpu.VMEM((CHUNK, 128), jnp.float32),
            pltpu.VMEM((16, 128), jnp.float32),
            pltpu.SemaphoreType.DMA,
        ],
    )(ta, tb, src3, dst3)


def _ew_body(att, src2, dst2, ew, s_v, d_v, a_v, b_v, o_v, sem_a, sem_b):
    c = lax.axis_index("c")
    s = lax.axis_index("s")
    w = c * 16 + s

    pltpu.sync_copy(src2.at[w], s_v)
    pltpu.sync_copy(dst2.at[w], d_v)

    def body(j, _):
        cp_a = pltpu.async_copy(att.at[s_v.at[j]], a_v, sem_a)
        cp_b = pltpu.async_copy(att.at[d_v.at[j]], b_v, sem_b)
        cp_a.wait()
        cp_b.wait()
        for k in range(CHUNK // 16):
            o_v[j, pl.ds(k * 16, 16)] = (a_v[pl.ds(k * 16, 16)] *
                                         b_v[pl.ds(k * 16, 16)])
        return 0

    lax.fori_loop(0, EW_NCH, body, 0)
    pltpu.sync_copy(o_v, ew.at[pl.ds(w * EW_NCH, EW_NCH)])


@jax.jit
def _ew(att, src2, dst2):
    return pl.kernel(
        _ew_body,
        out_type=jax.ShapeDtypeStruct((32 * EW_NCH, CHUNK), jnp.float32),
        mesh=_mesh(),
        scratch_types=[
            pltpu.VMEM((EW_NCH, CHUNK), jnp.int32),
            pltpu.VMEM((EW_NCH, CHUNK), jnp.int32),
            pltpu.VMEM((CHUNK,), jnp.float32),
            pltpu.VMEM((CHUNK,), jnp.float32),
            pltpu.VMEM((EW_NCH, CHUNK), jnp.float32),
            pltpu.SemaphoreType.DMA,
            pltpu.SemaphoreType.DMA,
        ],
    )(att, src2, dst2)


def _tca_body(acc1a, acc1b, xr, bf, gum, Wg, bg, Wib, bib,
              asn_o, u0a_o, u0b_o, g1_o):
    i = pl.program_id(0)
    agg = jnp.concatenate([acc1a[...], acc1b[...]], axis=1)
    z1 = jnp.maximum(agg @ Wg[...] + bg[...], 0.0)
    score = z1 @ Wib[...] + bib[...]
    m = jnp.max(score, axis=1, keepdims=True)
    e = jnp.exp(score - m)
    a1 = e / jnp.sum(e, axis=1, keepdims=True)
    t = a1 + gum[...]
    m2 = jnp.max(t, axis=1, keepdims=True)
    e2 = jnp.exp(t - m2)
    asn = e2 / jnp.sum(e2, axis=1, keepdims=True)
    asn_o[...] = asn
    x = xr[...]
    u0 = asn[:, 0:1] * x
    u0a_o[...] = u0[:, :128]
    u0b_o[...] = u0[:, 128:]
    oh = jnp.where(bf[...] == lax.broadcasted_iota(jnp.int32, (R, G), 1),
                   1.0, 0.0)
    contrib = lax.dot_general(oh, z1, (((0,), (0,)), ((), ())),
                              preferred_element_type=jnp.float32)

    @pl.when(i == 0)
    def _():
        g1_o[...] = jnp.zeros_like(g1_o)

    g1_o[...] += contrib


@jax.jit
def _tca(acc1a, acc1b, xp, bf, gum, Wg, bg, Wib, bib):
    row = lambda i: (i, 0)
    const = lambda i: (0, 0)
    return pl.pallas_call(
        _tca_body,
        grid=(GRID,),
        in_specs=[
            pl.BlockSpec((R, 128), row),
            pl.BlockSpec((R, 128), row),
            pl.BlockSpec((R, D), row),
            pl.BlockSpec((R, 1), row),
            pl.BlockSpec((R, 2), row),
            pl.BlockSpec((D, H), const),
            pl.BlockSpec((1, H), const),
            pl.BlockSpec((H, 2), const),
            pl.BlockSpec((1, 2), const),
        ],
        out_specs=[
            pl.BlockSpec((R, 2), row),
            pl.BlockSpec((R, 128), row),
            pl.BlockSpec((R, 128), row),
            pl.BlockSpec((G, H), const),
        ],
        out_shape=[
            jax.ShapeDtypeStruct((NP, 2), jnp.float32),
            jax.ShapeDtypeStruct((NP, 128), jnp.float32),
            jax.ShapeDtypeStruct((NP, 128), jnp.float32),
            jax.ShapeDtypeStruct((G, H), jnp.float32),
        ],
    )(acc1a, acc1b, xp, bf, gum, Wg, bg, Wib, bib)


def _tcb_body(accAa, accAb, acc1a, acc1b, asn, bf, Wctx, bctx, Wobj, bobj,
              g1r, Pr, yr, Wm1, bm1, Wm3, bm3, Wco1, bco1, Wco2, bco2,
              Wo1, bo1, Wo2, bo2, Wc1, bc1, Wc2, bc2, Wcs1, bcs1, Wcs2, bcs2,
              gM_o, gR_o, h1_o, hM_o, p1_o, pM_o, hco_o, hres_o, hcos_o, ys_o):
    i = pl.program_id(0)
    a = asn[...]
    accA = jnp.concatenate([accAa[...], accAb[...]], axis=1)
    acc1 = jnp.concatenate([acc1a[...], acc1b[...]], axis=1)
    aggO = a[:, 0:1] * accA
    aggC = a[:, 1:2] * (acc1 - accA)
    zM = jnp.maximum(aggO @ Wctx[...] + bctx[...], 0.0)
    zR = jnp.maximum(aggC @ Wobj[...] + bobj[...], 0.0)
    oh = jnp.where(bf[...] == lax.broadcasted_iota(jnp.int32, (R, G), 1),
                   1.0, 0.0)
    cM = lax.dot_general(oh, zM, (((0,), (0,)), ((), ())),
                         preferred_element_type=jnp.float32)
    cR = lax.dot_general(oh, zR, (((0,), (0,)), ((), ())),
                         preferred_element_type=jnp.float32)

    @pl.when(i == 0)
    def _():
        gM_o[...] = jnp.zeros_like(gM_o)
        gR_o[...] = jnp.zeros_like(gR_o)

    gM_o[...] += cM
    gR_o[...] += cR

    @pl.when(i == GRID - 1)
    def _():
        gM = gM_o[...]
        gR = gR_o[...]
        g1 = g1r[...]
        P = Pr[...]
        g_co = P @ gR + gM
        g_co_s = gR + P @ gM
        relu = lambda v: jnp.maximum(v, 0.0)
        h1_o[...] = g1 @ Wm1[...] + bm1[...]
        hM_o[...] = relu(gM @ Wo1[...] + bo1[...]) @ Wo2[...] + bo2[...]
        p1_o[...] = g1 @ Wm3[...] + bm3[...]
        pM_o[...] = gM @ Wm3[...] + bm3[...]
        hco_o[...] = relu(g_co @ Wco1[...] + bco1[...]) @ Wco2[...] + bco2[...]
        hres_o[...] = relu(gR @ Wc1[...] + bc1[...]) @ Wc2[...] + bc2[...]
        hcos_o[...] = relu(g_co_s @ Wcs1[...] + bcs1[...]) @ Wcs2[...] + bcs2[...]
        ys_o[...] = P @ yr[...]


@jax.jit
def _tcb(accAa, accAb, acc1a, acc1b, asn, bf, Wctx, bctx, Wobj, bobj,
         g1, P, yf, Wm1, bm1, Wm3, bm3, Wco1, bco1, Wco2, bco2,
         Wo1, bo1, Wo2, bo2, Wc1, bc1, Wc2, bc2, Wcs1, bcs1, Wcs2, bcs2):
    row = lambda i: (i, 0)
    const = lambda i: (0, 0)
    wspec = lambda shape: pl.BlockSpec(shape, const)
    return pl.pallas_call(
        _tcb_body,
        grid=(GRID,),
        in_specs=[
            pl.BlockSpec((R, 128), row),
            pl.BlockSpec((R, 128), row),
            pl.BlockSpec((R, 128), row),
            pl.BlockSpec((R, 128), row),
            pl.BlockSpec((R, 2), row),
            pl.BlockSpec((R, 1), row),
            wspec((D, H)), wspec((1, H)), wspec((D, H)), wspec((1, H)),
            wspec((G, H)), wspec((G, G)), wspec((G, 1)),
            wspec((H, C)), wspec((1, C)),
            wspec((H, H)), wspec((1, H)),
            wspec((H, H)), wspec((1, H)), wspec((H, C)), wspec((1, C)),
            wspec((H, H)), wspec((1, H)), wspec((H, C)), wspec((1, C)),
            wspec((H, H)), wspec((1, H)), wspec((H, C)), wspec((1, C)),
            wspec((H, H)), wspec((1, H)), wspec((H, C)), wspec((1, C)),
        ],
        out_specs=[
            wspec((G, H)), wspec((G, H)),
            wspec((G, C)), wspec((G, C)), wspec((G, H)), wspec((G, H)),
            wspec((G, C)), wspec((G, C)), wspec((G, C)), wspec((G, 1)),
        ],
        out_shape=[
            jax.ShapeDtypeStruct((G, H), jnp.float32),
            jax.ShapeDtypeStruct((G, H), jnp.float32),
            jax.ShapeDtypeStruct((G, C), jnp.float32),
            jax.ShapeDtypeStruct((G, C), jnp.float32),
            jax.ShapeDtypeStruct((G, H), jnp.float32),
            jax.ShapeDtypeStruct((G, H), jnp.float32),
            jax.ShapeDtypeStruct((G, C), jnp.float32),
            jax.ShapeDtypeStruct((G, C), jnp.float32),
            jax.ShapeDtypeStruct((G, C), jnp.float32),
            jax.ShapeDtypeStruct((G, 1), jnp.float32),
        ],
    )(accAa, accAb, acc1a, acc1b, asn, bf, Wctx, bctx, Wobj, bobj,
      g1, P, yf, Wm1, bm1, Wm3, bm3, Wco1, bco1, Wco2, bco2,
      Wo1, bo1, Wo2, bo2, Wc1, bc1, Wc2, bc2, Wcs1, bcs1, Wcs2, bcs2)


def kernel(x, edge_index, batch, y, W_g1, b_g1, W_ctx, b_ctx, W_obj, b_obj,
           W_ib, b_ib, W_m1, b_m1, W_m3, b_m3, W_co1, b_co1, W_co2, b_co2,
           W_o1, b_o1, W_o2, b_o2, W_c1, b_c1, W_c2, b_c2, W_cs1, b_cs1,
           W_cs2, b_cs2):
    f32 = jnp.float32

    xp = jnp.pad(x, ((0, NP - N), (0, 0)))
    xa = xp[:, :128]
    xb = xp[:, 128:]

    src = edge_index[0]
    dst = edge_index[1]
    src_p = jnp.pad(src, (0, EP - E))
    dst_p = jnp.pad(dst, (0, EP - E), constant_values=N)
    src3 = src_p.reshape(NT, NCH, CHUNK)
    dst3 = dst_p.reshape(NT, NCH, CHUNK)
    src2 = jnp.pad(src, (0, EP2 - E)).reshape(32, EW_NCH, CHUNK)
    dst2 = jnp.pad(dst, (0, EP2 - E)).reshape(32, EW_NCH, CHUNK)

    bf = jnp.pad(batch, (0, NP - N), constant_values=-1).reshape(NP, 1)

    u = jax.random.uniform(jax.random.key(42), (N, 2), minval=1e-10,
                           maxval=1.0)
    gum = -jnp.log(-jnp.log(u))
    gum = jnp.pad(gum, ((0, NP - N), (0, 0)))

    random_idx = jax.random.permutation(jax.random.key(7), G)
    P = (random_idx[:, None] == jnp.arange(G)[None, :]).astype(f32)
    yf = y.astype(f32)

    rb = lambda b: b.reshape(1, -1)

    acc1a, acc1b = _mp(xa, xb, src3, dst3)

    asn, u0a, u0b, g1 = _tca(
        acc1a, acc1b, xp, bf, gum, W_g1, rb(b_g1), W_ib, rb(b_ib))

    accAa, accAb = _mp(u0a, u0b, src3, dst3)

    att0 = asn[:, 0]
    ew = _ew(att0, src2, dst2)

    (gM, gR, h1, hM, p1, pM, hco, hres, hcos, ysf) = _tcb(
        accAa, accAb, acc1a, acc1b, asn, bf, W_ctx, rb(b_ctx), W_obj,
        rb(b_obj), g1, P, yf, W_m1, rb(b_m1), W_m3, rb(b_m3),
        W_co1, rb(b_co1), W_co2, rb(b_co2), W_o1, rb(b_o1), W_o2, rb(b_o2),
        W_c1, rb(b_c1), W_c2, rb(b_c2), W_cs1, rb(b_cs1), W_cs2, rb(b_cs2))

    assignment = asn[:N]
    edge_weight_o = ew.reshape(-1)[:E].reshape(E, 1)
    y_shuf = ysf[:, 0].astype(jnp.int32)

    return (h1, g1, hM, gM, p1, pM, assignment, hco, hres, edge_weight_o,
            hcos, y_shuf)
